# baseline (speedup 1.0000x reference)
import functools

import jax
import jax.numpy as jnp
from jax import lax
from jax.experimental import pallas as pl
from jax.experimental.pallas import tpu as pltpu

_N = 1000
_E = 1600000
_SIZES = (51200, 102400, 246400, 400000, 400000, 400000)
assert sum(_SIZES) == _E and all(sz % 128 == 0 for sz in _SIZES)
_NC = len(_SIZES)
_OFFS = tuple(sum(_SIZES[:i]) for i in range(_NC))


def _body(k_smem, ew_ref, ei_ref, ei_out, lp_out, *rest):
    bufs = rest[:_NC]
    in_sems = rest[_NC : 2 * _NC]
    out_sems = rest[2 * _NC :]

    def in_cp(i):
        return pltpu.make_async_copy(
            ei_ref.at[:, pl.ds(_OFFS[i], _SIZES[i])], bufs[i], in_sems[i]
        )

    def out_cp(i):
        return pltpu.make_async_copy(
            bufs[i], ei_out.at[:, pl.ds(_OFFS[i], _SIZES[i])], out_sems[i]
        )

    for i in range(_NC):
        in_cp(i).start()

    in_cp(0).wait()
    out_cp(0).start()

    r0 = ew_ref[0:1, :]
    r1 = ew_ref[1:2, :]
    x = jnp.where(k_smem[0] == 1, r1, r0)
    m = jnp.max(x)
    sum_x = jnp.sum(x)
    s = jnp.sum(jnp.exp(x - m))
    lp_out[0, 0] = sum_x - jnp.float32(_N) * m - jnp.float32(_N) * jnp.log(s)

    for i in range(1, _NC):
        in_cp(i).wait()
        out_cp(i).start()
    for i in range(_NC):
        out_cp(i).wait()


@jax.jit
def _run(edge_index, edge_weights, k):
    grid_spec = pltpu.PrefetchScalarGridSpec(
        num_scalar_prefetch=1,
        grid=(1,),
        in_specs=[
            pl.BlockSpec((2, _N), lambda i, k_ref: (0, 0)),
            pl.BlockSpec(memory_space=pl.ANY),
        ],
        out_specs=[
            pl.BlockSpec(memory_space=pl.ANY),
            pl.BlockSpec(memory_space=pltpu.SMEM),
        ],
        scratch_shapes=(
            [pltpu.VMEM((2, sz), jnp.int32) for sz in _SIZES]
            + [pltpu.SemaphoreType.DMA] * (2 * _NC)
        ),
    )
    ei_out, lp = pl.pallas_call(
        _body,
        grid_spec=grid_spec,
        out_shape=[
            jax.ShapeDtypeStruct((2, _E), jnp.int32),
            jax.ShapeDtypeStruct((1, 1), jnp.float32),
        ],
        compiler_params=pltpu.CompilerParams(
            dimension_semantics=("arbitrary",),
            vmem_limit_bytes=100 * 1024 * 1024,
        ),
    )(jnp.reshape(k, (1,)).astype(jnp.int32), edge_weights, edge_index)
    return ei_out, lp[0, 0]


def kernel(edge_index, edge_weights, n, num_sample, k):
    return _run(edge_index, edge_weights, k)


# speedup vs baseline: 1.0406x; 1.0406x over previous
import functools

import jax
import jax.numpy as jnp
from jax import lax
from jax.experimental import pallas as pl
from jax.experimental.pallas import tpu as pltpu

_N = 1000
_E = 1600000
_SIZES = (800000, 800000)
assert sum(_SIZES) == _E and all(sz % 128 == 0 for sz in _SIZES)
_NC = len(_SIZES)
_OFFS = tuple(sum(_SIZES[:i]) for i in range(_NC))


def _body(k_smem, ew_ref, ei_ref, ei_out, lp_out, *rest):
    bufs = rest[:_NC]
    in_sems = rest[_NC : 2 * _NC]
    out_sems = rest[2 * _NC :]

    def in_cp(i):
        return pltpu.make_async_copy(
            ei_ref.at[:, pl.ds(_OFFS[i], _SIZES[i])], bufs[i], in_sems[i]
        )

    def out_cp(i):
        return pltpu.make_async_copy(
            bufs[i], ei_out.at[:, pl.ds(_OFFS[i], _SIZES[i])], out_sems[i]
        )

    for i in range(_NC):
        in_cp(i).start()

    in_cp(0).wait()
    out_cp(0).start()

    r0 = ew_ref[0:1, :]
    r1 = ew_ref[1:2, :]
    x = jnp.where(k_smem[0] == 1, r1, r0)
    m = jnp.max(x)
    sum_x = jnp.sum(x)
    s = jnp.sum(jnp.exp(x - m))
    lp_out[0, 0] = sum_x - jnp.float32(_N) * m - jnp.float32(_N) * jnp.log(s)

    for i in range(1, _NC):
        in_cp(i).wait()
        out_cp(i).start()
    for i in range(_NC):
        out_cp(i).wait()


@jax.jit
def _run(edge_index, edge_weights, k):
    grid_spec = pltpu.PrefetchScalarGridSpec(
        num_scalar_prefetch=1,
        grid=(1,),
        in_specs=[
            pl.BlockSpec((2, _N), lambda i, k_ref: (0, 0)),
            pl.BlockSpec(memory_space=pl.ANY),
        ],
        out_specs=[
            pl.BlockSpec(memory_space=pl.ANY),
            pl.BlockSpec(memory_space=pltpu.SMEM),
        ],
        scratch_shapes=(
            [pltpu.VMEM((2, sz), jnp.int32) for sz in _SIZES]
            + [pltpu.SemaphoreType.DMA] * (2 * _NC)
        ),
    )
    ei_out, lp = pl.pallas_call(
        _body,
        grid_spec=grid_spec,
        out_shape=[
            jax.ShapeDtypeStruct((2, _E), jnp.int32),
            jax.ShapeDtypeStruct((1, 1), jnp.float32),
        ],
        compiler_params=pltpu.CompilerParams(
            dimension_semantics=("arbitrary",),
            vmem_limit_bytes=100 * 1024 * 1024,
        ),
    )(jnp.reshape(k, (1,)).astype(jnp.int32), edge_weights, edge_index)
    return ei_out, lp[0, 0]


def kernel(edge_index, edge_weights, n, num_sample, k):
    return _run(edge_index, edge_weights, k)
